# Initial kernel scaffold; baseline (speedup 1.0000x reference)
import functools

import jax
import jax.numpy as jnp
from jax import lax
from jax.experimental import pallas as pl
from jax.experimental.pallas import tpu as pltpu
from jax.experimental.pallas import tpu_sc as plsc

B = 4096
G = 100
OD = 16
NW = 32
BT = B // NW
NCHUNK = BT // 16


def _body(at_hbm, tab_hbm, out_hbm, a_buf, tab_buf, scr, out_buf):
    wid = lax.axis_index("s") * 2 + lax.axis_index("c")
    pltpu.sync_copy(at_hbm.at[wid], a_buf)
    pltpu.sync_copy(tab_hbm, tab_buf)

    iota16x = lax.iota(jnp.int32, 16) * 16
    zeros_i = jnp.zeros((16,), jnp.int32)

    def chunk_body(c, carry):
        def g_body(g, carry2):
            base = g * 128 + c * 16
            va0 = a_buf[pl.ds(base, 16)]
            va1 = a_buf[pl.ds(base + G * BT, 16)]
            va2 = a_buf[pl.ds(base + 2 * G * BT, 16)]
            vmin = jnp.minimum(jnp.minimum(va0, va1), va2)
            vmax = jnp.maximum(jnp.maximum(va0, va1), va2)
            vmid = jnp.maximum(jnp.minimum(va0, va1),
                               jnp.minimum(jnp.maximum(va0, va1), va2))
            c0 = vmin
            c1 = vmid - vmin
            c2 = vmax - vmid
            pmin = jnp.where(va0 == vmin, jnp.int32(1),
                             jnp.where(va1 == vmin, jnp.int32(2), jnp.int32(4)))
            pmax = jnp.where(va2 == vmax, jnp.int32(4),
                             jnp.where(va1 == vmax, jnp.int32(2), jnp.int32(1)))
            gbase = g * 128
            idx_mid = gbase + 112 - pmin * 16
            idx_max = gbase + pmax * 16
            idx7 = zeros_i + (gbase + 112)
            for l in range(16):
                s7 = plsc.load_gather(tab_buf, [idx7 + l])
                smid = plsc.load_gather(tab_buf, [idx_mid + l])
                smax = plsc.load_gather(tab_buf, [idx_max + l])
                scr[pl.ds(l * 16, 16)] = c0 * s7 + c1 * smid + c2 * smax
            for j in range(16):
                row = plsc.load_gather(scr, [iota16x + j])
                out_buf[pl.ds(j * (G * OD) + g * OD, 16)] = row
            return carry2

        lax.fori_loop(0, G, g_body, carry)
        b0 = wid * BT + c * 16
        pltpu.sync_copy(out_buf, out_hbm.at[pl.ds(b0 * G * OD, 16 * G * OD)])
        return carry

    lax.fori_loop(0, NCHUNK, chunk_body, 0)


@jax.jit
def kernel(X, params):
    at = X.reshape(NW, BT, G, 3).transpose(0, 3, 2, 1).reshape(NW, 3 * G * BT)
    tab = params.reshape(G * 8 * OD)
    run = pl.kernel(
        _body,
        out_type=jax.ShapeDtypeStruct((B * G * OD,), jnp.float32),
        mesh=plsc.VectorSubcoreMesh(core_axis_name="c", subcore_axis_name="s"),
        scratch_types=[
            pltpu.VMEM((3 * G * BT,), jnp.float32),
            pltpu.VMEM((G * 8 * OD,), jnp.float32),
            pltpu.VMEM((256,), jnp.float32),
            pltpu.VMEM((16 * G * OD,), jnp.float32),
        ],
    )
    out = run(at, tab)
    return out.reshape(B, G * OD)


# speedup vs baseline: 38.9607x; 38.9607x over previous
import functools

import jax
import jax.numpy as jnp
from jax import lax
from jax.experimental import pallas as pl
from jax.experimental.pallas import tpu as pltpu
from jax.experimental.pallas import tpu_sc as plsc

B = 4096
G = 100
OD = 16
NW = 32
BT = B // NW
NCHUNK = BT // 16


def _body(at_hbm, tab_hbm, out_hbm, a_buf, tab_buf, scr, out_buf):
    wid = lax.axis_index("s") * 2 + lax.axis_index("c")
    pltpu.sync_copy(at_hbm.at[wid], a_buf)
    pltpu.sync_copy(tab_hbm, tab_buf)

    iota16x = lax.iota(jnp.int32, 16) * 16
    zeros_i = jnp.zeros((16,), jnp.int32)

    def chunk_body(c, carry):
        def g_body(g, carry2):
            base = g * 128 + c * 16
            va0 = a_buf[pl.ds(base, 16)]
            va1 = a_buf[pl.ds(base + G * BT, 16)]
            va2 = a_buf[pl.ds(base + 2 * G * BT, 16)]
            vmin = jnp.minimum(jnp.minimum(va0, va1), va2)
            vmax = jnp.maximum(jnp.maximum(va0, va1), va2)
            vmid = jnp.maximum(jnp.minimum(va0, va1),
                               jnp.minimum(jnp.maximum(va0, va1), va2))
            c0 = vmin
            c1 = vmid - vmin
            c2 = vmax - vmid
            pmin = jnp.where(va0 == vmin, jnp.int32(1),
                             jnp.where(va1 == vmin, jnp.int32(2), jnp.int32(4)))
            pmax = jnp.where(va2 == vmax, jnp.int32(4),
                             jnp.where(va1 == vmax, jnp.int32(2), jnp.int32(1)))
            gbase = g * 128
            idx_mid = gbase + 112 - pmin * 16
            idx_max = gbase + pmax * 16
            idx7 = zeros_i + (gbase + 112)
            for l in range(16):
                s7 = plsc.load_gather(tab_buf, [idx7 + l])
                smid = plsc.load_gather(tab_buf, [idx_mid + l])
                smax = plsc.load_gather(tab_buf, [idx_max + l])
                scr[pl.ds(l * 16, 16)] = c0 * s7 + c1 * smid + c2 * smax
            for j in range(16):
                row = plsc.load_gather(scr, [iota16x + j])
                out_buf[pl.ds(j * (G * OD) + g * OD, 16)] = row
            return carry2

        lax.fori_loop(0, G, g_body, carry)
        b0 = wid * BT + c * 16
        pltpu.sync_copy(out_buf, out_hbm.at[pl.ds(b0 * G * OD, 16 * G * OD)])
        return carry

    lax.fori_loop(0, NCHUNK, chunk_body, 0)


@jax.jit
def kernel(X, params):
    at = X.reshape(NW, BT, G, 3).transpose(0, 3, 2, 1).reshape(NW, 3 * G * BT)
    tab = params.reshape(G * 8 * OD)
    run = pl.kernel(
        _body,
        out_type=jax.ShapeDtypeStruct((B * G * OD,), jnp.float32),
        mesh=plsc.VectorSubcoreMesh(core_axis_name="c", subcore_axis_name="s"),
        compiler_params=pltpu.CompilerParams(needs_layout_passes=False),
        scratch_types=[
            pltpu.VMEM((3 * G * BT,), jnp.float32),
            pltpu.VMEM((G * 8 * OD,), jnp.float32),
            pltpu.VMEM((256,), jnp.float32),
            pltpu.VMEM((16 * G * OD,), jnp.float32),
        ],
    )
    out = run(at, tab)
    return out.reshape(B, G * OD)


# speedup vs baseline: 63.9315x; 1.6409x over previous
import functools

import jax
import jax.numpy as jnp
from jax import lax
from jax.experimental import pallas as pl
from jax.experimental.pallas import tpu as pltpu
from jax.experimental.pallas import tpu_sc as plsc

B = 4096
G = 100
OD = 16
NW = 32
BT = B // NW
NCHUNK = BT // 16


def _body(at_hbm, tab_hbm, out_hbm, a_buf, tab_buf, scr, out_buf):
    wid = lax.axis_index("s") * 2 + lax.axis_index("c")
    pltpu.sync_copy(at_hbm.at[wid], a_buf)
    pltpu.sync_copy(tab_hbm, tab_buf)

    iota16x = lax.iota(jnp.int32, 16) * 17
    zeros_i = jnp.zeros((16,), jnp.int32)

    def chunk_body(c, carry):
        def g_body(g, carry2):
            base = g * 128 + c * 16
            va0 = a_buf[pl.ds(base, 16)]
            va1 = a_buf[pl.ds(base + G * BT, 16)]
            va2 = a_buf[pl.ds(base + 2 * G * BT, 16)]
            vmin = jnp.minimum(jnp.minimum(va0, va1), va2)
            vmax = jnp.maximum(jnp.maximum(va0, va1), va2)
            vmid = jnp.maximum(jnp.minimum(va0, va1),
                               jnp.minimum(jnp.maximum(va0, va1), va2))
            c0 = vmin
            c1 = vmid - vmin
            c2 = vmax - vmid
            pmin = jnp.where(va0 == vmin, jnp.int32(1),
                             jnp.where(va1 == vmin, jnp.int32(2), jnp.int32(4)))
            pmax = jnp.where(va2 == vmax, jnp.int32(4),
                             jnp.where(va1 == vmax, jnp.int32(2), jnp.int32(1)))
            gbase = g * 136
            idx_mid = gbase + 119 - pmin * 17
            idx_max = gbase + pmax * 17
            idx7 = zeros_i + (gbase + 119)
            for l in range(16):
                s7 = plsc.load_gather(tab_buf, [idx7 + l])
                smid = plsc.load_gather(tab_buf, [idx_mid + l])
                smax = plsc.load_gather(tab_buf, [idx_max + l])
                scr[pl.ds(l * 17, 16)] = c0 * s7 + c1 * smid + c2 * smax
            for j in range(16):
                row = plsc.load_gather(scr, [iota16x + j])
                out_buf[pl.ds(j * (G * OD) + g * OD, 16)] = row
            return carry2

        lax.fori_loop(0, G, g_body, carry)
        b0 = wid * BT + c * 16
        pltpu.sync_copy(out_buf, out_hbm.at[pl.ds(b0 * G * OD, 16 * G * OD)])
        return carry

    lax.fori_loop(0, NCHUNK, chunk_body, 0)


@jax.jit
def kernel(X, params):
    at = X.reshape(NW, BT, G, 3).transpose(0, 3, 2, 1).reshape(NW, 3 * G * BT)
    tab = jnp.pad(params.reshape(G * 8, OD), ((0, 0), (0, 1))).reshape(G * 8 * 17)
    run = pl.kernel(
        _body,
        out_type=jax.ShapeDtypeStruct((B * G * OD,), jnp.float32),
        mesh=plsc.VectorSubcoreMesh(core_axis_name="c", subcore_axis_name="s"),
        compiler_params=pltpu.CompilerParams(needs_layout_passes=False),
        scratch_types=[
            pltpu.VMEM((3 * G * BT,), jnp.float32),
            pltpu.VMEM((G * 8 * 17,), jnp.float32),
            pltpu.VMEM((272,), jnp.float32),
            pltpu.VMEM((16 * G * OD,), jnp.float32),
        ],
    )
    out = run(at, tab)
    return out.reshape(B, G * OD)


# speedup vs baseline: 65.7086x; 1.0278x over previous
import functools

import jax
import jax.numpy as jnp
from jax import lax
from jax.experimental import pallas as pl
from jax.experimental.pallas import tpu as pltpu
from jax.experimental.pallas import tpu_sc as plsc

B = 4096
G = 100
OD = 16
NW = 32
BT = B // NW
NCHUNK = BT // 16


def _body(at_hbm, tab_hbm, out_hbm, a_buf, tab_buf, scr, out_buf):
    wid = lax.axis_index("s") * 2 + lax.axis_index("c")
    pltpu.sync_copy(at_hbm.at[wid], a_buf)
    pltpu.sync_copy(tab_hbm, tab_buf)

    iota16x = lax.iota(jnp.int32, 16) * 17
    zeros_i = jnp.zeros((16,), jnp.int32)

    def chunk_body(c, carry):
        def g_body(g, carry2):
            base = g * 128 + c * 16
            va0 = a_buf[pl.ds(base, 16)]
            va1 = a_buf[pl.ds(base + G * BT, 16)]
            va2 = a_buf[pl.ds(base + 2 * G * BT, 16)]
            vmin = jnp.minimum(jnp.minimum(va0, va1), va2)
            vmax = jnp.maximum(jnp.maximum(va0, va1), va2)
            vmid = jnp.maximum(jnp.minimum(va0, va1),
                               jnp.minimum(jnp.maximum(va0, va1), va2))
            c0 = vmin
            c1 = vmid - vmin
            c2 = vmax - vmid
            pmin = jnp.where(va0 == vmin, jnp.int32(1),
                             jnp.where(va1 == vmin, jnp.int32(2), jnp.int32(4)))
            pmax = jnp.where(va2 == vmax, jnp.int32(4),
                             jnp.where(va1 == vmax, jnp.int32(2), jnp.int32(1)))
            gbase = g * 136
            idx_mid = gbase + 119 - pmin * 17
            idx_max = gbase + pmax * 17
            row7 = tab_buf[pl.ds(gbase + 119, 16)]
            for l in range(16):
                s7 = jnp.broadcast_to(row7[l], (16,))
                smid = plsc.load_gather(tab_buf, [idx_mid + l])
                smax = plsc.load_gather(tab_buf, [idx_max + l])
                scr[pl.ds(l * 17, 16)] = c0 * s7 + c1 * smid + c2 * smax
            for j in range(16):
                row = plsc.load_gather(scr, [iota16x + j])
                out_buf[pl.ds(j * (G * OD) + g * OD, 16)] = row
            return carry2

        lax.fori_loop(0, G, g_body, carry)
        b0 = wid * BT + c * 16
        pltpu.sync_copy(out_buf, out_hbm.at[pl.ds(b0 * G * OD, 16 * G * OD)])
        return carry

    lax.fori_loop(0, NCHUNK, chunk_body, 0)


@jax.jit
def kernel(X, params):
    at = X.reshape(NW, BT, G, 3).transpose(0, 3, 2, 1).reshape(NW, 3 * G * BT)
    tab = jnp.pad(params.reshape(G * 8, OD), ((0, 0), (0, 1))).reshape(G * 8 * 17)
    run = pl.kernel(
        _body,
        out_type=jax.ShapeDtypeStruct((B * G * OD,), jnp.float32),
        mesh=plsc.VectorSubcoreMesh(core_axis_name="c", subcore_axis_name="s"),
        compiler_params=pltpu.CompilerParams(needs_layout_passes=False),
        scratch_types=[
            pltpu.VMEM((3 * G * BT,), jnp.float32),
            pltpu.VMEM((G * 8 * 17,), jnp.float32),
            pltpu.VMEM((272,), jnp.float32),
            pltpu.VMEM((16 * G * OD,), jnp.float32),
        ],
    )
    out = run(at, tab)
    return out.reshape(B, G * OD)


# speedup vs baseline: 89.6444x; 1.3643x over previous
import jax
import jax.numpy as jnp
from jax import lax
from jax.experimental import pallas as pl
from jax.experimental.pallas import tpu as pltpu
from jax.experimental.pallas import tpu_sc as plsc

B = 4096
G = 100
OD = 16
NW = 32
BT = B // NW
NCHUNK = BT // 16
ROWP = G * OD + 1
OUT_HALF = 16 * ROWP


def _body(at_hbm, tab_hbm, out_hbm, a_buf, tab_buf, out_buf, sem0, sem1):
    wid = lax.axis_index("s") * 2 + lax.axis_index("c")
    pltpu.sync_copy(at_hbm.at[wid], a_buf)
    pltpu.sync_copy(tab_hbm, tab_buf)

    iota = lax.iota(jnp.int32, 16)
    row_scatter = iota * ROWP

    def do_chunk(c, par, sem):
        def g_loop(_):
            @plsc.parallel_loop(0, G, unroll=4)
            def g_body(g):
                base = g * BT + c * 16
                va0 = a_buf[pl.ds(base, 16)]
                va1 = a_buf[pl.ds(base + G * BT, 16)]
                va2 = a_buf[pl.ds(base + 2 * G * BT, 16)]
                vmin = jnp.minimum(jnp.minimum(va0, va1), va2)
                vmax = jnp.maximum(jnp.maximum(va0, va1), va2)
                vmid = jnp.maximum(jnp.minimum(va0, va1),
                                   jnp.minimum(jnp.maximum(va0, va1), va2))
                c0 = vmin
                c1 = vmid - vmin
                c2 = vmax - vmid
                pmin = jnp.where(va0 == vmin, jnp.int32(1),
                                 jnp.where(va1 == vmin, jnp.int32(2),
                                           jnp.int32(4)))
                pmax = jnp.where(va2 == vmax, jnp.int32(4),
                                 jnp.where(va1 == vmax, jnp.int32(2),
                                           jnp.int32(1)))
                gbase = g * 136
                idx_mid = gbase + 119 - pmin * 17
                idx_max = gbase + pmax * 17
                row7 = tab_buf[pl.ds(gbase + 119, 16)]
                sc_base = row_scatter + (par + g * OD)
                for l in range(16):
                    s7 = jnp.broadcast_to(row7[l], (16,))
                    smid = plsc.load_gather(tab_buf, [idx_mid + l])
                    smax = plsc.load_gather(tab_buf, [idx_max + l])
                    o = c0 * s7 + c1 * smid + c2 * smax
                    plsc.store_scatter(out_buf, [sc_base + l], o)

        g_loop(None)
        b0 = wid * BT + c * 16
        pltpu.async_copy(
            out_buf.at[pl.ds(par, OUT_HALF)],
            out_hbm.at[pl.ds(b0 * ROWP, OUT_HALF)],
            sem,
        )

    def wait_half(c, par, sem):
        b0 = wid * BT + c * 16
        pltpu.make_async_copy(
            out_buf.at[pl.ds(par, OUT_HALF)],
            out_hbm.at[pl.ds(b0 * ROWP, OUT_HALF)],
            sem,
        ).wait()

    def pair_body(cc, carry):
        c_even = cc * 2
        c_odd = cc * 2 + 1

        @pl.when(cc >= 1)
        def _w0():
            wait_half(c_even - 2, 0, sem0)

        do_chunk(c_even, 0, sem0)

        @pl.when(cc >= 1)
        def _w1():
            wait_half(c_odd - 2, OUT_HALF, sem1)

        do_chunk(c_odd, OUT_HALF, sem1)
        return carry

    lax.fori_loop(0, NCHUNK // 2, pair_body, 0)
    wait_half(NCHUNK - 2, 0, sem0)
    wait_half(NCHUNK - 1, OUT_HALF, sem1)


@jax.jit
def kernel(X, params):
    at = X.reshape(NW, BT, G, 3).transpose(0, 3, 2, 1).reshape(NW, 3 * G * BT)
    tab = jnp.pad(params.reshape(G * 8, OD), ((0, 0), (0, 1))).reshape(G * 8 * 17)
    run = pl.kernel(
        _body,
        out_type=jax.ShapeDtypeStruct((B * ROWP,), jnp.float32),
        mesh=plsc.VectorSubcoreMesh(core_axis_name="c", subcore_axis_name="s"),
        compiler_params=pltpu.CompilerParams(needs_layout_passes=False),
        scratch_types=[
            pltpu.VMEM((3 * G * BT,), jnp.float32),
            pltpu.VMEM((G * 8 * 17,), jnp.float32),
            pltpu.VMEM((2 * OUT_HALF,), jnp.float32),
            pltpu.SemaphoreType.DMA,
            pltpu.SemaphoreType.DMA,
        ],
    )
    out = run(at, tab)
    return out.reshape(B, ROWP)[:, : G * OD]
